# Initial kernel scaffold; baseline (speedup 1.0000x reference)
import jax
import jax.numpy as jnp
from jax.experimental import pallas as pl


def kernel(x, table, W, b):
    raise NotImplementedError("write your pallas kernel here")


# speedup vs baseline: 3.0797x; 3.0797x over previous
import functools

import jax
import jax.numpy as jnp
from jax import lax
from jax.experimental import pallas as pl
from jax.experimental.pallas import tpu as pltpu
from jax.experimental.pallas import tpu_sc as plsc

VOCAB = 1000
EMBED = 32
MAXLEN = 200
NCLASS = 100
BATCH = 4096

N_TOK = BATCH * MAXLEN
NC, NS = 2, 16
NW = NC * NS
ROWS_PER_W = BATCH // NW
ROWS_PER_CHUNK = 8
NCHUNK = ROWS_PER_W // ROWS_PER_CHUNK
TOK_PER_CHUNK = ROWS_PER_CHUNK * MAXLEN
NGROUP = TOK_PER_CHUNK // 16


def _sc_gather(x_flat, table_flat):
    mesh = plsc.VectorSubcoreMesh(core_axis_name="c", subcore_axis_name="s")

    @functools.partial(
        pl.kernel,
        out_type=jax.ShapeDtypeStruct((BATCH, MAXLEN * EMBED), jnp.float32),
        mesh=mesh,
        scratch_types=[
            pltpu.VMEM((TOK_PER_CHUNK,), jnp.int32),
            pltpu.VMEM((VOCAB * EMBED,), jnp.float32),
            pltpu.VMEM((ROWS_PER_CHUNK, MAXLEN * EMBED), jnp.float32),
        ],
        compiler_params=pltpu.CompilerParams(needs_layout_passes=False),
    )
    def k(idx_hbm, table_hbm, out_hbm, idx_v, table_v, out_v):
        wid = lax.axis_index("s") * NC + lax.axis_index("c")
        pltpu.sync_copy(table_hbm, table_v)
        row0 = wid * ROWS_PER_W
        lane = lax.iota(jnp.int32, 16)

        def chunk_body(ci, carry):
            r0 = row0 + ci * ROWS_PER_CHUNK
            pltpu.sync_copy(
                idx_hbm.at[pl.ds(r0 * MAXLEN, TOK_PER_CHUNK)], idx_v
            )

            def group_body(g, c2):
                t16 = g * 16 + lane
                x16 = idx_v[pl.ds(g * 16, 16)]
                r16 = t16 // MAXLEN
                c0 = (t16 - r16 * MAXLEN) * EMBED
                f0 = x16 * EMBED
                for e in range(EMBED):
                    vals = plsc.load_gather(table_v, [f0 + e])
                    plsc.store_scatter(out_v, [r16, c0 + e], vals)
                return c2

            lax.fori_loop(0, NGROUP, group_body, 0)
            pltpu.sync_copy(out_v, out_hbm.at[pl.ds(r0, ROWS_PER_CHUNK), :])
            return carry

        lax.fori_loop(0, NCHUNK, chunk_body, 0)

    return k(x_flat, table_flat)


def _tc_matmul(emb, W, b):
    TB = 256

    def mm(e_ref, w_ref, b_ref, o_ref):
        o_ref[...] = (
            jnp.dot(
                e_ref[...].astype(jnp.bfloat16),
                w_ref[...],
                preferred_element_type=jnp.float32,
            )
            + b_ref[...]
        )

    return pl.pallas_call(
        mm,
        grid=(BATCH // TB,),
        in_specs=[
            pl.BlockSpec((TB, MAXLEN * EMBED), lambda i: (i, 0)),
            pl.BlockSpec((MAXLEN * EMBED, NCLASS), lambda i: (0, 0)),
            pl.BlockSpec((1, NCLASS), lambda i: (0, 0)),
        ],
        out_specs=pl.BlockSpec((TB, NCLASS), lambda i: (i, 0)),
        out_shape=jax.ShapeDtypeStruct((BATCH, NCLASS), jnp.float32),
    )(emb, W.astype(jnp.bfloat16), b.reshape(1, NCLASS))


def kernel(x, table, W, b):
    x_flat = x.reshape(N_TOK).astype(jnp.int32)
    table_flat = table.reshape(VOCAB * EMBED)
    emb = _sc_gather(x_flat, table_flat)
    return _tc_matmul(emb, W, b)
